# baseline (speedup 1.0000x reference)
import functools

import numpy as np
import jax
import jax.numpy as jnp
from jax import lax
from jax.experimental import pallas as pl
from jax.experimental.pallas import tpu as pltpu
from jax.experimental.pallas import tpu_sc as plsc

N = 10000
F = 128
HID = 128
C = 40
CP = 64
E = 320000

NC = 2
NS = 16
NW = NC * NS
CH = 128
NROW = 10112
RPT = NROW // NS
DUMP0 = 10048
EPW = 10240
CH_PER_W = EPW // CH
IBLK = 8
NBLK = CH_PER_W // IBLK
EPAD = EPW * NW
ERW = E // NW


def _unpack_perm(width):
    perm = np.zeros((width,), np.int32)
    for j in range(width // 32):
        for l in range(16):
            perm[32 * j + l] = 32 * j + 2 * l
            perm[32 * j + 16 + l] = 32 * j + 2 * l + 1
    return perm

PERM1 = _unpack_perm(HID)
PERM2 = _unpack_perm(CP)


def _pack_bf16(a):
    n, m = a.shape
    b = a.astype(jnp.bfloat16).reshape(n, m // 2, 2)
    return jax.lax.bitcast_convert_type(b, jnp.int32)


def _mm1_body(x_ref, wl_ref, wr_ref, b1_ref, xl_ref, xr_ref):
    xb = x_ref[...]
    xl_ref[...] = jnp.dot(xb, wl_ref[...], preferred_element_type=jnp.float32)
    xr_ref[...] = (
        jnp.dot(xb, wr_ref[...], preferred_element_type=jnp.float32)
        + b1_ref[...]
    )


def _mm1(x, W1l, W1rp, b1p):
    bm = 1000
    return pl.pallas_call(
        _mm1_body,
        grid=(N // bm,),
        in_specs=[
            pl.BlockSpec((bm, F), lambda i: (i, 0)),
            pl.BlockSpec((F, HID), lambda i: (0, 0)),
            pl.BlockSpec((F, HID), lambda i: (0, 0)),
            pl.BlockSpec((1, HID), lambda i: (0, 0)),
        ],
        out_specs=[
            pl.BlockSpec((bm, HID), lambda i: (i, 0)),
            pl.BlockSpec((bm, HID), lambda i: (i, 0)),
        ],
        out_shape=[
            jax.ShapeDtypeStruct((N, HID), jnp.float32),
            jax.ShapeDtypeStruct((N, HID), jnp.float32),
        ],
    )(x, W1l, W1rp, b1p.reshape(1, HID))


def _sc_agg_body(wi, with_deg, *refs):
    if with_deg:
        (tbl, srch, dsth, wh, zx, zd, aggp, degp,
         src_v, dst_v, w_v, rowsi0, rowsi1, rowsf, ones_v, accx, accd,
         sem0, sem1) = refs
    else:
        (tbl, srch, dsth, wh, zx, aggp,
         src_v, dst_v, w_v, rowsi0, rowsi1, rowsf, accx,
         sem0, sem1) = refs
    c = lax.axis_index("c")
    s = lax.axis_index("s")
    wid = s * NC + c
    r0 = pl.multiple_of(s * RPT, 8)

    pltpu.sync_copy(zx.at[pl.ds(r0, RPT)], accx.at[pl.ds(r0, RPT)])
    if with_deg:
        pltpu.sync_copy(zd.at[pl.ds(r0, RPT)], accd.at[pl.ds(r0, RPT)])

        def init_ones(i, _):
            ones_v[i, :] = jnp.full((16,), 1.0, jnp.float32)
            return 0
        lax.fori_loop(0, CH, init_ones, 0)
    plsc.subcore_barrier()

    himask = jnp.full((16,), -65536, jnp.int32)

    def proc(g, rowsi_v):
        def grp(q, _):
            wv = w_v[g, pl.ds(q * 16, 16)]
            for l in range(16):
                bw = lax.gather(
                    wv, jnp.full((16, 1), l, jnp.int32),
                    lax.GatherDimensionNumbers(
                        offset_dims=(), collapsed_slice_dims=(0,),
                        start_index_map=(0,)),
                    (1,), mode=lax.GatherScatterMode.PROMISE_IN_BOUNDS)
                e = q * 16 + l
                for j in range(wi // 16):
                    v = rowsi_v[e, pl.ds(j * 16, 16)]
                    lo = plsc.bitcast(lax.shift_left(v, 16), jnp.float32)
                    hi = plsc.bitcast(lax.bitwise_and(v, himask),
                                      jnp.float32)
                    rowsf[e, pl.ds(32 * j, 16)] = lo * bw
                    rowsf[e, pl.ds(32 * j + 16, 16)] = hi * bw
            return 0
        lax.fori_loop(0, CH // 16, grp, 0)
        pltpu.sync_copy(rowsf, accx.at[dst_v.at[g]], add=True)
        if with_deg:
            pltpu.sync_copy(ones_v, accd.at[dst_v.at[g]], add=True)

    def block(b, _):
        crow = wid * CH_PER_W + b * IBLK
        pltpu.sync_copy(srch.at[pl.ds(crow, IBLK)], src_v)
        pltpu.sync_copy(dsth.at[pl.ds(crow, IBLK)], dst_v)
        pltpu.sync_copy(wh.at[pl.ds(crow, IBLK)], w_v)
        pltpu.async_copy(tbl.at[src_v.at[0]], rowsi0, sem0)

        def pair(i, _):
            g0 = i * 2
            pltpu.async_copy(tbl.at[src_v.at[g0 + 1]], rowsi1, sem1)
            pltpu.make_async_copy(tbl.at[src_v.at[g0]], rowsi0, sem0).wait()
            proc(g0, rowsi0)

            @pl.when(g0 + 2 < IBLK)
            def _():
                pltpu.async_copy(tbl.at[src_v.at[g0 + 2]], rowsi0, sem0)
            pltpu.make_async_copy(
                tbl.at[src_v.at[g0 + 1]], rowsi1, sem1).wait()
            proc(g0 + 1, rowsi1)
            return 0
        lax.fori_loop(0, IBLK // 2, pair, 0)
        return 0
    lax.fori_loop(0, NBLK, block, 0)
    plsc.subcore_barrier()

    pltpu.sync_copy(accx.at[pl.ds(r0, RPT)], aggp.at[c, pl.ds(r0, RPT)])
    if with_deg:
        pltpu.sync_copy(accd.at[pl.ds(r0, RPT)], degp.at[c, pl.ds(r0, RPT)])


def _sc_agg(wi, with_deg):
    mesh = plsc.VectorSubcoreMesh(core_axis_name="c", subcore_axis_name="s")
    wf = 2 * wi
    out_type = [jax.ShapeDtypeStruct((NC, NROW, wf), jnp.float32)]
    scratch = [
        pltpu.VMEM((IBLK, CH), jnp.int32),
        pltpu.VMEM((IBLK, CH), jnp.int32),
        pltpu.VMEM((IBLK, CH), jnp.float32),
        pltpu.VMEM((CH, wi), jnp.int32),
        pltpu.VMEM((CH, wi), jnp.int32),
        pltpu.VMEM((CH, wf), jnp.float32),
    ]
    if with_deg:
        out_type.append(jax.ShapeDtypeStruct((NC, NROW, 16), jnp.float32))
        scratch.append(pltpu.VMEM((CH, 16), jnp.float32))
    scratch.append(pltpu.VMEM_SHARED((NROW, wf), jnp.float32))
    if with_deg:
        scratch.append(pltpu.VMEM_SHARED((NROW, 16), jnp.float32))
    scratch.append(pltpu.SemaphoreType.DMA)
    scratch.append(pltpu.SemaphoreType.DMA)
    return pl.kernel(
        functools.partial(_sc_agg_body, wi, with_deg),
        out_type=out_type,
        mesh=mesh,
        scratch_types=scratch,
        compiler_params=pltpu.CompilerParams(
            use_tc_tiling_on_sc=False, needs_layout_passes=False),
    )


def _mid_body(a0_ref, a1_ref, d0_ref, d1_ref, xr_ref, wl_ref, wr_ref, b2_ref,
              hl_ref, hr_ref):
    agg = a0_ref[...] + a1_ref[...]
    deg = d0_ref[:, 0:1] + d1_ref[:, 0:1]
    rdeg = 1.0 / jnp.maximum(deg, 1.0)
    h = jnp.maximum(agg * rdeg + xr_ref[...], 0.0)
    hl_ref[...] = jnp.dot(h, wl_ref[...], preferred_element_type=jnp.float32)
    hr_ref[...] = (
        jnp.dot(h, wr_ref[...], preferred_element_type=jnp.float32)
        + b2_ref[...]
    )


def _mid(a0, a1, d0, d1, xr, W2lp, W2rp, b2p):
    bm = 1000
    return pl.pallas_call(
        _mid_body,
        grid=(N // bm,),
        in_specs=[
            pl.BlockSpec((bm, HID), lambda i: (i, 0)),
            pl.BlockSpec((bm, HID), lambda i: (i, 0)),
            pl.BlockSpec((bm, 16), lambda i: (i, 0)),
            pl.BlockSpec((bm, 16), lambda i: (i, 0)),
            pl.BlockSpec((bm, HID), lambda i: (i, 0)),
            pl.BlockSpec((HID, CP), lambda i: (0, 0)),
            pl.BlockSpec((HID, CP), lambda i: (0, 0)),
            pl.BlockSpec((1, CP), lambda i: (0, 0)),
        ],
        out_specs=[
            pl.BlockSpec((bm, CP), lambda i: (i, 0)),
            pl.BlockSpec((bm, CP), lambda i: (i, 0)),
        ],
        out_shape=[
            jax.ShapeDtypeStruct((N, CP), jnp.float32),
            jax.ShapeDtypeStruct((N, CP), jnp.float32),
        ],
    )(a0, a1, d0, d1, xr, W2lp, W2rp, b2p)


def _fin_body(a0_ref, a1_ref, d0_ref, d1_ref, hr_ref, msk_ref, p_ref,
              out_ref):
    agg = a0_ref[...] + a1_ref[...]
    deg = d0_ref[:, 0:1] + d1_ref[:, 0:1]
    rdeg = 1.0 / jnp.maximum(deg, 1.0)
    logits = agg * rdeg + hr_ref[...]
    masked = logits + msk_ref[...]
    m = jnp.max(masked, axis=1, keepdims=True)
    lse = jnp.log(jnp.sum(jnp.exp(masked - m), axis=1, keepdims=True)) + m
    out_ref[...] = jnp.dot(logits - lse, p_ref[...],
                           preferred_element_type=jnp.float32)


def _fin(a0, a1, d0, d1, hr, msk, pmat):
    bm = 1000
    return pl.pallas_call(
        _fin_body,
        grid=(N // bm,),
        in_specs=[
            pl.BlockSpec((bm, CP), lambda i: (i, 0)),
            pl.BlockSpec((bm, CP), lambda i: (i, 0)),
            pl.BlockSpec((bm, 16), lambda i: (i, 0)),
            pl.BlockSpec((bm, 16), lambda i: (i, 0)),
            pl.BlockSpec((bm, CP), lambda i: (i, 0)),
            pl.BlockSpec((1, CP), lambda i: (0, 0)),
            pl.BlockSpec((CP, 48), lambda i: (0, 0)),
        ],
        out_specs=pl.BlockSpec((bm, 48), lambda i: (i, 0)),
        out_shape=jax.ShapeDtypeStruct((N, 48), jnp.float32),
    )(a0, a1, d0, d1, hr, msk, pmat)


@jax.jit
def kernel(x, edge_index, edge_weight, W1l, W1r, b1, W2l, W2r, b2):
    src = edge_index[0]
    dst = edge_index[1]
    padw = EPW - ERW
    dpad = jnp.broadcast_to(
        DUMP0 + (jnp.arange(padw, dtype=jnp.int32) % 64), (NW, padw))
    srcp = jnp.concatenate(
        [src.reshape(NW, ERW), jnp.zeros((NW, padw), jnp.int32)],
        axis=1).reshape(EPAD // CH, CH)
    dstp = jnp.concatenate(
        [dst.reshape(NW, ERW), dpad], axis=1).reshape(EPAD // CH, CH)
    wp = jnp.concatenate(
        [edge_weight.reshape(NW, ERW), jnp.zeros((NW, padw), jnp.float32)],
        axis=1).reshape(EPAD // CH, CH)

    zx = jnp.zeros((NROW, HID), jnp.float32)
    zd = jnp.zeros((NROW, 16), jnp.float32)
    zc = jnp.zeros((NROW, CP), jnp.float32)

    W1rp = W1r[:, PERM1]
    b1p = b1[PERM1]
    W2l64 = jnp.pad(W2l, ((0, 0), (0, CP - C)))
    W2r64 = jnp.pad(W2r, ((0, 0), (0, CP - C)))
    W2lp = W2l64[PERM1, :]
    W2rp = W2r64[PERM1, :][:, PERM2]
    b2p = jnp.pad(b2, (0, CP - C))[PERM2].reshape(1, CP)
    msk = jnp.where(PERM2 < C, 0.0, -1e30).astype(jnp.float32).reshape(1, CP)
    pmat = (PERM2[:, None] == np.arange(48)[None, :]).astype(np.float32)

    xl, xr = _mm1(x, W1l, W1rp, b1p)
    aggp, degp = _sc_agg(HID // 2, True)(
        _pack_bf16(xl), srcp, dstp, wp, zx, zd)
    hl, hr = _mid(aggp[0], aggp[1], degp[0], degp[1], xr, W2lp, W2rp, b2p)
    (agg2p,) = _sc_agg(CP // 2, False)(_pack_bf16(hl), srcp, dstp, wp, zc)
    out = _fin(agg2p[0], agg2p[1], degp[0], degp[1], hr, msk, jnp.asarray(pmat))
    return out[:, :C]


# speedup vs baseline: 1.1915x; 1.1915x over previous
import functools

import numpy as np
import jax
import jax.numpy as jnp
from jax import lax
from jax.experimental import pallas as pl
from jax.experimental.pallas import tpu as pltpu
from jax.experimental.pallas import tpu_sc as plsc

N = 10000
F = 128
HID = 128
C = 40
CP = 64
E = 320000

NC = 2
NS = 16
NW = NC * NS
CH = 128
NROW = 10112
RPT = NROW // NS
DUMP0 = 10048
EPW = 10240
CH_PER_W = EPW // CH
IBLK = 8
NBLK = CH_PER_W // IBLK
EPAD = EPW * NW
ERW = E // NW


def _unpack_perm(width):
    perm = np.zeros((width,), np.int32)
    for j in range(width // 32):
        for l in range(16):
            perm[32 * j + l] = 32 * j + 2 * l
            perm[32 * j + 16 + l] = 32 * j + 2 * l + 1
    return perm

PERM1 = _unpack_perm(HID)
PERM2 = _unpack_perm(CP)


def _pack_bf16(a):
    n, m = a.shape
    b = a.astype(jnp.bfloat16).reshape(n, m // 2, 2)
    return jax.lax.bitcast_convert_type(b, jnp.int32)


def _mm1_body(x_ref, wl_ref, wr_ref, b1_ref, xl_ref, xr_ref):
    xb = x_ref[...]
    xl_ref[...] = jnp.dot(xb, wl_ref[...], preferred_element_type=jnp.float32)
    xr_ref[...] = (
        jnp.dot(xb, wr_ref[...], preferred_element_type=jnp.float32)
        + b1_ref[...]
    )


def _mm1(x, W1l, W1rp, b1p):
    bm = 1000
    return pl.pallas_call(
        _mm1_body,
        grid=(N // bm,),
        in_specs=[
            pl.BlockSpec((bm, F), lambda i: (i, 0)),
            pl.BlockSpec((F, HID), lambda i: (0, 0)),
            pl.BlockSpec((F, HID), lambda i: (0, 0)),
            pl.BlockSpec((1, HID), lambda i: (0, 0)),
        ],
        out_specs=[
            pl.BlockSpec((bm, HID), lambda i: (i, 0)),
            pl.BlockSpec((bm, HID), lambda i: (i, 0)),
        ],
        out_shape=[
            jax.ShapeDtypeStruct((N, HID), jnp.float32),
            jax.ShapeDtypeStruct((N, HID), jnp.float32),
        ],
    )(x, W1l, W1rp, b1p.reshape(1, HID))


def _sc_agg_body(wi, npass, with_deg, *refs):
    n_in = npass + 4 + (1 if with_deg else 0)
    tbls = refs[:npass]
    (srch, dsth, wh, zx) = refs[npass:npass + 4]
    zd = refs[npass + 4] if with_deg else None
    aggs = refs[n_in:n_in + npass]
    degp = refs[n_in + npass] if with_deg else None
    sc = n_in + npass + (1 if with_deg else 0)
    if with_deg:
        (src_v, dst_v, w_v, rowsi0, rowsi1, rowsf, ones_v,
         tbl_sh, accx, accd, sem0, sem1) = refs[sc:]
    else:
        (src_v, dst_v, w_v, rowsi0, rowsi1, rowsf,
         tbl_sh, accx, sem0, sem1) = refs[sc:]
    c = lax.axis_index("c")
    s = lax.axis_index("s")
    wid = s * NC + c
    r0 = pl.multiple_of(s * RPT, 8)

    himask = jnp.full((16,), -65536, jnp.int32)

    def proc(g, rowsi_v, first_pass):
        def grp(q, _):
            wv = w_v[g, pl.ds(q * 16, 16)]
            for l in range(16):
                bw = lax.gather(
                    wv, jnp.full((16, 1), l, jnp.int32),
                    lax.GatherDimensionNumbers(
                        offset_dims=(), collapsed_slice_dims=(0,),
                        start_index_map=(0,)),
                    (1,), mode=lax.GatherScatterMode.PROMISE_IN_BOUNDS)
                e = q * 16 + l
                for j in range(wi // 16):
                    v = rowsi_v[e, pl.ds(j * 16, 16)]
                    lo = plsc.bitcast(lax.shift_left(v, 16), jnp.float32)
                    hi = plsc.bitcast(lax.bitwise_and(v, himask),
                                      jnp.float32)
                    rowsf[e, pl.ds(32 * j, 16)] = lo * bw
                    rowsf[e, pl.ds(32 * j + 16, 16)] = hi * bw
            return 0
        lax.fori_loop(0, CH // 16, grp, 0)
        pltpu.sync_copy(rowsf, accx.at[dst_v.at[g]], add=True)
        if first_pass and with_deg:
            pltpu.sync_copy(ones_v, accd.at[dst_v.at[g]], add=True)

    for p in range(npass):
        first = p == 0
        pltpu.sync_copy(zx.at[pl.ds(r0, RPT)], accx.at[pl.ds(r0, RPT)])
        pltpu.sync_copy(tbls[p].at[pl.ds(r0, RPT)],
                        tbl_sh.at[pl.ds(r0, RPT)])
        if first and with_deg:
            pltpu.sync_copy(zd.at[pl.ds(r0, RPT)], accd.at[pl.ds(r0, RPT)])

            def init_ones(i, _):
                ones_v[i, :] = jnp.full((16,), 1.0, jnp.float32)
                return 0
            lax.fori_loop(0, CH, init_ones, 0)
        plsc.subcore_barrier()

        def block(b, _):
            crow = wid * CH_PER_W + b * IBLK
            pltpu.sync_copy(srch.at[pl.ds(crow, IBLK)], src_v)
            pltpu.sync_copy(dsth.at[pl.ds(crow, IBLK)], dst_v)
            pltpu.sync_copy(wh.at[pl.ds(crow, IBLK)], w_v)
            pltpu.async_copy(tbl_sh.at[src_v.at[0]], rowsi0, sem0)

            def pair(i, _):
                g0 = i * 2
                pltpu.async_copy(tbl_sh.at[src_v.at[g0 + 1]], rowsi1, sem1)
                pltpu.make_async_copy(
                    tbl_sh.at[src_v.at[g0]], rowsi0, sem0).wait()
                proc(g0, rowsi0, first)

                @pl.when(g0 + 2 < IBLK)
                def _():
                    pltpu.async_copy(
                        tbl_sh.at[src_v.at[g0 + 2]], rowsi0, sem0)
                pltpu.make_async_copy(
                    tbl_sh.at[src_v.at[g0 + 1]], rowsi1, sem1).wait()
                proc(g0 + 1, rowsi1, first)
                return 0
            lax.fori_loop(0, IBLK // 2, pair, 0)
            return 0
        lax.fori_loop(0, NBLK, block, 0)
        plsc.subcore_barrier()

        pltpu.sync_copy(accx.at[pl.ds(r0, RPT)], aggs[p].at[c, pl.ds(r0, RPT)])
        if first and with_deg:
            pltpu.sync_copy(accd.at[pl.ds(r0, RPT)],
                            degp.at[c, pl.ds(r0, RPT)])


def _sc_agg(wi, npass, with_deg):
    mesh = plsc.VectorSubcoreMesh(core_axis_name="c", subcore_axis_name="s")
    wf = 2 * wi
    out_type = [jax.ShapeDtypeStruct((NC, NROW, wf), jnp.float32)
                for _ in range(npass)]
    scratch = [
        pltpu.VMEM((IBLK, CH), jnp.int32),
        pltpu.VMEM((IBLK, CH), jnp.int32),
        pltpu.VMEM((IBLK, CH), jnp.float32),
        pltpu.VMEM((CH, wi), jnp.int32),
        pltpu.VMEM((CH, wi), jnp.int32),
        pltpu.VMEM((CH, wf), jnp.float32),
    ]
    if with_deg:
        out_type.append(jax.ShapeDtypeStruct((NC, NROW, 16), jnp.float32))
        scratch.append(pltpu.VMEM((CH, 16), jnp.float32))
    scratch.append(pltpu.VMEM_SHARED((NROW, wi), jnp.int32))
    scratch.append(pltpu.VMEM_SHARED((NROW, wf), jnp.float32))
    if with_deg:
        scratch.append(pltpu.VMEM_SHARED((NROW, 16), jnp.float32))
    scratch.append(pltpu.SemaphoreType.DMA)
    scratch.append(pltpu.SemaphoreType.DMA)
    return pl.kernel(
        functools.partial(_sc_agg_body, wi, npass, with_deg),
        out_type=out_type,
        mesh=mesh,
        scratch_types=scratch,
        compiler_params=pltpu.CompilerParams(
            use_tc_tiling_on_sc=False, needs_layout_passes=False),
    )


def _mid_body(a00_ref, a01_ref, a10_ref, a11_ref, d0_ref, d1_ref, xr_ref,
              wl_ref, wr_ref, b2_ref, hl_ref, hr_ref):
    agg = jnp.concatenate(
        [a00_ref[...] + a01_ref[...], a10_ref[...] + a11_ref[...]], axis=1)
    deg = d0_ref[:, 0:1] + d1_ref[:, 0:1]
    rdeg = 1.0 / jnp.maximum(deg, 1.0)
    h = jnp.maximum(agg * rdeg + xr_ref[...], 0.0)
    hl_ref[...] = jnp.dot(h, wl_ref[...], preferred_element_type=jnp.float32)
    hr_ref[...] = (
        jnp.dot(h, wr_ref[...], preferred_element_type=jnp.float32)
        + b2_ref[...]
    )


def _mid(a00, a01, a10, a11, d0, d1, xr, W2lp, W2rp, b2p):
    bm = 1000
    return pl.pallas_call(
        _mid_body,
        grid=(N // bm,),
        in_specs=[
            pl.BlockSpec((bm, HID // 2), lambda i: (i, 0)),
            pl.BlockSpec((bm, HID // 2), lambda i: (i, 0)),
            pl.BlockSpec((bm, HID // 2), lambda i: (i, 0)),
            pl.BlockSpec((bm, HID // 2), lambda i: (i, 0)),
            pl.BlockSpec((bm, 16), lambda i: (i, 0)),
            pl.BlockSpec((bm, 16), lambda i: (i, 0)),
            pl.BlockSpec((bm, HID), lambda i: (i, 0)),
            pl.BlockSpec((HID, CP), lambda i: (0, 0)),
            pl.BlockSpec((HID, CP), lambda i: (0, 0)),
            pl.BlockSpec((1, CP), lambda i: (0, 0)),
        ],
        out_specs=[
            pl.BlockSpec((bm, CP), lambda i: (i, 0)),
            pl.BlockSpec((bm, CP), lambda i: (i, 0)),
        ],
        out_shape=[
            jax.ShapeDtypeStruct((N, CP), jnp.float32),
            jax.ShapeDtypeStruct((N, CP), jnp.float32),
        ],
    )(a00, a01, a10, a11, d0, d1, xr, W2lp, W2rp, b2p)


def _fin_body(a0_ref, a1_ref, d0_ref, d1_ref, hr_ref, msk_ref, p_ref,
              out_ref):
    agg = a0_ref[...] + a1_ref[...]
    deg = d0_ref[:, 0:1] + d1_ref[:, 0:1]
    rdeg = 1.0 / jnp.maximum(deg, 1.0)
    logits = agg * rdeg + hr_ref[...]
    masked = logits + msk_ref[...]
    m = jnp.max(masked, axis=1, keepdims=True)
    lse = jnp.log(jnp.sum(jnp.exp(masked - m), axis=1, keepdims=True)) + m
    out_ref[...] = jnp.dot(logits - lse, p_ref[...],
                           preferred_element_type=jnp.float32)


def _fin(a0, a1, d0, d1, hr, msk, pmat):
    bm = 1000
    return pl.pallas_call(
        _fin_body,
        grid=(N // bm,),
        in_specs=[
            pl.BlockSpec((bm, CP), lambda i: (i, 0)),
            pl.BlockSpec((bm, CP), lambda i: (i, 0)),
            pl.BlockSpec((bm, 16), lambda i: (i, 0)),
            pl.BlockSpec((bm, 16), lambda i: (i, 0)),
            pl.BlockSpec((bm, CP), lambda i: (i, 0)),
            pl.BlockSpec((1, CP), lambda i: (0, 0)),
            pl.BlockSpec((CP, 48), lambda i: (0, 0)),
        ],
        out_specs=pl.BlockSpec((bm, 48), lambda i: (i, 0)),
        out_shape=jax.ShapeDtypeStruct((N, 48), jnp.float32),
    )(a0, a1, d0, d1, hr, msk, pmat)


@jax.jit
def kernel(x, edge_index, edge_weight, W1l, W1r, b1, W2l, W2r, b2):
    src = edge_index[0]
    dst = edge_index[1]
    padw = EPW - ERW
    dpad = jnp.broadcast_to(
        DUMP0 + (jnp.arange(padw, dtype=jnp.int32) % 64), (NW, padw))
    srcp = jnp.concatenate(
        [src.reshape(NW, ERW), jnp.zeros((NW, padw), jnp.int32)],
        axis=1).reshape(EPAD // CH, CH)
    dstp = jnp.concatenate(
        [dst.reshape(NW, ERW), dpad], axis=1).reshape(EPAD // CH, CH)
    wp = jnp.concatenate(
        [edge_weight.reshape(NW, ERW), jnp.zeros((NW, padw), jnp.float32)],
        axis=1).reshape(EPAD // CH, CH)

    zx = jnp.zeros((NROW, CP), jnp.float32)
    zd = jnp.zeros((NROW, 16), jnp.float32)

    W1rp = W1r[:, PERM1]
    b1p = b1[PERM1]
    W2l64 = jnp.pad(W2l, ((0, 0), (0, CP - C)))
    W2r64 = jnp.pad(W2r, ((0, 0), (0, CP - C)))
    W2lp = W2l64[PERM1, :]
    W2rp = W2r64[PERM1, :][:, PERM2]
    b2p = jnp.pad(b2, (0, CP - C))[PERM2].reshape(1, CP)
    msk = jnp.where(PERM2 < C, 0.0, -1e30).astype(jnp.float32).reshape(1, CP)
    pmat = (PERM2[:, None] == np.arange(48)[None, :]).astype(np.float32)

    xl, xr = _mm1(x, W1l, W1rp, b1p)
    xlt = _pack_bf16(jnp.pad(xl, ((0, NROW - N), (0, 0))))
    tbl0 = xlt[:, :HID // 4]
    tbl1 = xlt[:, HID // 4:]
    agg0, agg1, degp = _sc_agg(HID // 4, 2, True)(
        tbl0, tbl1, srcp, dstp, wp, zx, zd)
    hl, hr = _mid(agg0[0], agg0[1], agg1[0], agg1[1], degp[0], degp[1],
                  xr, W2lp, W2rp, b2p)
    hlt = _pack_bf16(jnp.pad(hl, ((0, NROW - N), (0, 0))))
    (agg2p,) = _sc_agg(CP // 2, 1, False)(hlt, srcp, dstp, wp, zx)
    out = _fin(agg2p[0], agg2p[1], degp[0], degp[1], hr, msk, jnp.asarray(pmat))
    return out[:, :C]


# speedup vs baseline: 1.4448x; 1.2126x over previous
import functools

import numpy as np
import jax
import jax.numpy as jnp
from jax import lax
from jax.experimental import pallas as pl
from jax.experimental.pallas import tpu as pltpu
from jax.experimental.pallas import tpu_sc as plsc

N = 10000
F = 128
HID = 128
C = 40
CP = 64
E = 320000

NC = 2
NS = 16
NW = NC * NS
CH = 128
NROW = 10112
RPT = NROW // NS
DUMP0 = 10048
EPW = 10240
CH_PER_W = EPW // CH
IBLK = 16
NBLK = CH_PER_W // IBLK
EPAD = EPW * NW
ERW = E // NW


def _unpack_perm(width):
    perm = np.zeros((width,), np.int32)
    for j in range(width // 32):
        for l in range(16):
            perm[32 * j + l] = 32 * j + 2 * l
            perm[32 * j + 16 + l] = 32 * j + 2 * l + 1
    return perm

PERM1 = _unpack_perm(HID)
PERM2 = _unpack_perm(CP)


def _pack_bf16(a):
    n, m = a.shape
    b = a.astype(jnp.bfloat16).reshape(n, m // 2, 2)
    return jax.lax.bitcast_convert_type(b, jnp.int32)


def _mm1_body(x_ref, wl_ref, wr_ref, b1_ref, xl_ref, xr_ref):
    xb = x_ref[...]
    xl_ref[...] = jnp.dot(xb, wl_ref[...], preferred_element_type=jnp.float32)
    xr_ref[...] = (
        jnp.dot(xb, wr_ref[...], preferred_element_type=jnp.float32)
        + b1_ref[...]
    )


def _mm1(x, W1l, W1rp, b1p):
    bm = 1000
    return pl.pallas_call(
        _mm1_body,
        grid=(N // bm,),
        in_specs=[
            pl.BlockSpec((bm, F), lambda i: (i, 0)),
            pl.BlockSpec((F, HID), lambda i: (0, 0)),
            pl.BlockSpec((F, HID), lambda i: (0, 0)),
            pl.BlockSpec((1, HID), lambda i: (0, 0)),
        ],
        out_specs=[
            pl.BlockSpec((bm, HID), lambda i: (i, 0)),
            pl.BlockSpec((bm, HID), lambda i: (i, 0)),
        ],
        out_shape=[
            jax.ShapeDtypeStruct((N, HID), jnp.float32),
            jax.ShapeDtypeStruct((N, HID), jnp.float32),
        ],
    )(x, W1l, W1rp, b1p.reshape(1, HID))


def _sc_agg_body(wi, npass, with_deg, *refs):
    wf = 2 * wi
    tbls = refs[:npass]
    (srch, dsth, wh, zx) = refs[npass:npass + 4]
    aggs = refs[npass + 4:npass + 4 + npass]
    (src_v, dst_v, w_v, rowsi0, rowsi1, rowsf0, rowsf1,
     tbl_sh, accx, semg0, semg1, sems0, sems1) = refs[npass + 4 + npass:]
    c = lax.axis_index("c")
    s = lax.axis_index("s")
    wid = s * NC + c
    r0 = pl.multiple_of(s * RPT, 8)

    himask = jnp.full((16,), -65536, jnp.int32)

    if with_deg:
        def init_ones(i, _):
            rowsf0[i, pl.ds(wf, 16)] = jnp.full((16,), 1.0, jnp.float32)
            rowsf1[i, pl.ds(wf, 16)] = jnp.full((16,), 1.0, jnp.float32)
            return 0
        lax.fori_loop(0, CH, init_ones, 0)

    def mult(g, rowsi_v, rowsf_v):
        def grp(q, _):
            wv = w_v[g, pl.ds(q * 16, 16)]
            for l in range(16):
                bw = lax.gather(
                    wv, jnp.full((16, 1), l, jnp.int32),
                    lax.GatherDimensionNumbers(
                        offset_dims=(), collapsed_slice_dims=(0,),
                        start_index_map=(0,)),
                    (1,), mode=lax.GatherScatterMode.PROMISE_IN_BOUNDS)
                e = q * 16 + l
                for j in range(wi // 16):
                    v = rowsi_v[e, pl.ds(j * 16, 16)]
                    lo = plsc.bitcast(lax.shift_left(v, 16), jnp.float32)
                    hi = plsc.bitcast(lax.bitwise_and(v, himask),
                                      jnp.float32)
                    rowsf_v[e, pl.ds(32 * j, 16)] = lo * bw
                    rowsf_v[e, pl.ds(32 * j + 16, 16)] = hi * bw
            return 0
        lax.fori_loop(0, CH // 16, grp, 0)

    for p in range(npass):
        pltpu.sync_copy(zx.at[pl.ds(r0, RPT)], accx.at[pl.ds(r0, RPT)])
        pltpu.sync_copy(tbls[p].at[pl.ds(r0, RPT)],
                        tbl_sh.at[pl.ds(r0, RPT)])
        plsc.subcore_barrier()

        def block(b, _):
            crow = wid * CH_PER_W + b * IBLK
            pltpu.sync_copy(srch.at[pl.ds(crow, IBLK)], src_v)
            pltpu.sync_copy(dsth.at[pl.ds(crow, IBLK)], dst_v)
            pltpu.sync_copy(wh.at[pl.ds(crow, IBLK)], w_v)
            pltpu.async_copy(tbl_sh.at[src_v.at[0]], rowsi0, semg0)
            pltpu.async_copy(tbl_sh.at[src_v.at[1]], rowsi1, semg1)

            def pair(i, _):
                g0 = i * 2
                pltpu.make_async_copy(
                    tbl_sh.at[src_v.at[g0]], rowsi0, semg0).wait()

                @pl.when(i > 0)
                def _():
                    pltpu.make_async_copy(
                        rowsf0, accx.at[dst_v.at[g0 - 2]], sems0).wait()
                mult(g0, rowsi0, rowsf0)

                @pl.when(g0 + 2 < IBLK)
                def _():
                    pltpu.async_copy(
                        tbl_sh.at[src_v.at[g0 + 2]], rowsi0, semg0)
                pltpu.async_copy(
                    rowsf0, accx.at[dst_v.at[g0]], sems0, add=True)

                pltpu.make_async_copy(
                    tbl_sh.at[src_v.at[g0 + 1]], rowsi1, semg1).wait()

                @pl.when(i > 0)
                def _():
                    pltpu.make_async_copy(
                        rowsf1, accx.at[dst_v.at[g0 - 1]], sems1).wait()
                mult(g0 + 1, rowsi1, rowsf1)

                @pl.when(g0 + 3 < IBLK)
                def _():
                    pltpu.async_copy(
                        tbl_sh.at[src_v.at[g0 + 3]], rowsi1, semg1)
                pltpu.async_copy(
                    rowsf1, accx.at[dst_v.at[g0 + 1]], sems1, add=True)
                return 0
            lax.fori_loop(0, IBLK // 2, pair, 0)
            pltpu.make_async_copy(
                rowsf0, accx.at[dst_v.at[IBLK - 2]], sems0).wait()
            pltpu.make_async_copy(
                rowsf1, accx.at[dst_v.at[IBLK - 1]], sems1).wait()
            return 0
        lax.fori_loop(0, NBLK, block, 0)
        plsc.subcore_barrier()

        pltpu.sync_copy(accx.at[pl.ds(r0, RPT)], aggs[p].at[c, pl.ds(r0, RPT)])


def _sc_agg(wi, npass, with_deg):
    mesh = plsc.VectorSubcoreMesh(core_axis_name="c", subcore_axis_name="s")
    wa = 2 * wi + (16 if with_deg else 0)
    out_type = [jax.ShapeDtypeStruct((NC, NROW, wa), jnp.float32)
                for _ in range(npass)]
    scratch = [
        pltpu.VMEM((IBLK, CH), jnp.int32),
        pltpu.VMEM((IBLK, CH), jnp.int32),
        pltpu.VMEM((IBLK, CH), jnp.float32),
        pltpu.VMEM((CH, wi), jnp.int32),
        pltpu.VMEM((CH, wi), jnp.int32),
        pltpu.VMEM((CH, wa), jnp.float32),
        pltpu.VMEM((CH, wa), jnp.float32),
        pltpu.VMEM_SHARED((NROW, wi), jnp.int32),
        pltpu.VMEM_SHARED((NROW, wa), jnp.float32),
        pltpu.SemaphoreType.DMA,
        pltpu.SemaphoreType.DMA,
        pltpu.SemaphoreType.DMA,
        pltpu.SemaphoreType.DMA,
    ]
    return pl.kernel(
        functools.partial(_sc_agg_body, wi, npass, with_deg),
        out_type=out_type,
        mesh=mesh,
        scratch_types=scratch,
        compiler_params=pltpu.CompilerParams(
            use_tc_tiling_on_sc=False, needs_layout_passes=False),
    )


def _mid_body(a00_ref, a01_ref, a10_ref, a11_ref, xr_ref,
              wl_ref, wr_ref, b2_ref, hl_ref, hr_ref):
    aggl = a00_ref[...] + a01_ref[...]
    aggh = a10_ref[...] + a11_ref[...]
    agg = jnp.concatenate([aggl[:, :HID // 2], aggh[:, :HID // 2]], axis=1)
    deg = aggl[:, HID // 2:HID // 2 + 1]
    rdeg = 1.0 / jnp.maximum(deg, 1.0)
    h = jnp.maximum(agg * rdeg + xr_ref[...], 0.0)
    hl_ref[...] = jnp.dot(h, wl_ref[...], preferred_element_type=jnp.float32)
    hr_ref[...] = (
        jnp.dot(h, wr_ref[...], preferred_element_type=jnp.float32)
        + b2_ref[...]
    )


def _mid(a00, a01, a10, a11, xr, W2lp, W2rp, b2p):
    bm = 1000
    wa = HID // 2 + 16
    return pl.pallas_call(
        _mid_body,
        grid=(N // bm,),
        in_specs=[
            pl.BlockSpec((bm, wa), lambda i: (i, 0)),
            pl.BlockSpec((bm, wa), lambda i: (i, 0)),
            pl.BlockSpec((bm, wa), lambda i: (i, 0)),
            pl.BlockSpec((bm, wa), lambda i: (i, 0)),
            pl.BlockSpec((bm, HID), lambda i: (i, 0)),
            pl.BlockSpec((HID, CP), lambda i: (0, 0)),
            pl.BlockSpec((HID, CP), lambda i: (0, 0)),
            pl.BlockSpec((1, CP), lambda i: (0, 0)),
        ],
        out_specs=[
            pl.BlockSpec((bm, CP), lambda i: (i, 0)),
            pl.BlockSpec((bm, CP), lambda i: (i, 0)),
        ],
        out_shape=[
            jax.ShapeDtypeStruct((N, CP), jnp.float32),
            jax.ShapeDtypeStruct((N, CP), jnp.float32),
        ],
    )(a00, a01, a10, a11, xr, W2lp, W2rp, b2p)


def _fin_body(a0_ref, a1_ref, d0_ref, d1_ref, hr_ref, msk_ref, p_ref,
              out_ref):
    agg = a0_ref[...] + a1_ref[...]
    deg = d0_ref[:, CP:CP + 1] + d1_ref[:, CP:CP + 1]
    rdeg = 1.0 / jnp.maximum(deg, 1.0)
    logits = agg * rdeg + hr_ref[...]
    masked = logits + msk_ref[...]
    m = jnp.max(masked, axis=1, keepdims=True)
    lse = jnp.log(jnp.sum(jnp.exp(masked - m), axis=1, keepdims=True)) + m
    out_ref[...] = jnp.dot(logits - lse, p_ref[...],
                           preferred_element_type=jnp.float32)


def _fin(a0, a1, d0, d1, hr, msk, pmat):
    bm = 1000
    return pl.pallas_call(
        _fin_body,
        grid=(N // bm,),
        in_specs=[
            pl.BlockSpec((bm, CP), lambda i: (i, 0)),
            pl.BlockSpec((bm, CP), lambda i: (i, 0)),
            pl.BlockSpec((bm, CP + 16), lambda i: (i, 0)),
            pl.BlockSpec((bm, CP + 16), lambda i: (i, 0)),
            pl.BlockSpec((bm, CP), lambda i: (i, 0)),
            pl.BlockSpec((1, CP), lambda i: (0, 0)),
            pl.BlockSpec((CP, 48), lambda i: (0, 0)),
        ],
        out_specs=pl.BlockSpec((bm, 48), lambda i: (i, 0)),
        out_shape=jax.ShapeDtypeStruct((N, 48), jnp.float32),
    )(a0, a1, d0, d1, hr, msk, pmat)


@jax.jit
def kernel(x, edge_index, edge_weight, W1l, W1r, b1, W2l, W2r, b2):
    src = edge_index[0]
    dst = edge_index[1]
    padw = EPW - ERW
    dpad = jnp.broadcast_to(
        DUMP0 + (jnp.arange(padw, dtype=jnp.int32) % 64), (NW, padw))
    srcp = jnp.concatenate(
        [src.reshape(NW, ERW), jnp.zeros((NW, padw), jnp.int32)],
        axis=1).reshape(EPAD // CH, CH)
    dstp = jnp.concatenate(
        [dst.reshape(NW, ERW), dpad], axis=1).reshape(EPAD // CH, CH)
    wp = jnp.concatenate(
        [edge_weight.reshape(NW, ERW), jnp.zeros((NW, padw), jnp.float32)],
        axis=1).reshape(EPAD // CH, CH)

    zx80 = jnp.zeros((NROW, CP + 16), jnp.float32)
    zx64 = jnp.zeros((NROW, CP), jnp.float32)

    W1rp = W1r[:, PERM1]
    b1p = b1[PERM1]
    W2l64 = jnp.pad(W2l, ((0, 0), (0, CP - C)))
    W2r64 = jnp.pad(W2r, ((0, 0), (0, CP - C)))
    W2lp = W2l64[PERM1, :]
    W2rp = W2r64[PERM1, :][:, PERM2]
    b2p = jnp.pad(b2, (0, CP - C))[PERM2].reshape(1, CP)
    msk = jnp.where(PERM2 < C, 0.0, -1e30).astype(jnp.float32).reshape(1, CP)
    pmat = (PERM2[:, None] == np.arange(48)[None, :]).astype(np.float32)

    xl, xr = _mm1(x, W1l, W1rp, b1p)
    xlt = _pack_bf16(jnp.pad(xl, ((0, NROW - N), (0, 0))))
    tbl0 = xlt[:, :HID // 4]
    tbl1 = xlt[:, HID // 4:]
    agg0, agg1 = _sc_agg(HID // 4, 2, True)(
        tbl0, tbl1, srcp, dstp, wp, zx80)
    hl, hr = _mid(agg0[0], agg0[1], agg1[0], agg1[1],
                  xr, W2lp, W2rp, b2p)
    hlt = _pack_bf16(jnp.pad(hl, ((0, NROW - N), (0, 0))))
    (agg2p,) = _sc_agg(CP // 2, 1, False)(hlt, srcp, dstp, wp, zx64)
    out = _fin(agg2p[0], agg2p[1], agg0[0], agg0[1], hr, msk,
               jnp.asarray(pmat))
    return out[:, :C]


# speedup vs baseline: 1.5533x; 1.0751x over previous
import functools

import numpy as np
import jax
import jax.numpy as jnp
from jax import lax
from jax.experimental import pallas as pl
from jax.experimental.pallas import tpu as pltpu
from jax.experimental.pallas import tpu_sc as plsc

N = 10000
F = 128
HID = 128
C = 40
CP = 64
E = 320000

NC = 2
NS = 16
NW = NC * NS
CH = 128
NROW = 10112
RPT = NROW // NS
DUMP0 = 10048
EPW = 10240
CH_PER_W = EPW // CH
IBLK = 16
NBLK = CH_PER_W // IBLK
EPAD = EPW * NW
ERW = E // NW


def _unpack_perm(width):
    half = width // 2
    perm = np.zeros((width,), np.int32)
    for j in range(half // 16):
        for l in range(16):
            perm[32 * j + l] = 16 * j + l
            perm[32 * j + 16 + l] = half + 16 * j + l
    return perm

PERM1 = _unpack_perm(HID)
PERM2 = _unpack_perm(CP)


def _tc_pack(a):
    k = a.shape[1] // 2
    bits = jax.lax.bitcast_convert_type(a, jnp.int32) + 0x8000
    lo = jax.lax.shift_right_logical(bits[:, :k], 16)
    hi = jax.lax.bitwise_and(bits[:, k:], -65536)
    return jax.lax.bitwise_or(lo, hi)


def _mm1_body(x_ref, wl_ref, wr_ref, b1_ref, t0_ref, t1_ref, xr_ref):
    xb = x_ref[...]
    xl = jnp.dot(xb, wl_ref[...], preferred_element_type=jnp.float32)
    xli = _tc_pack(xl)
    t0_ref[...] = xli[:, :HID // 4]
    t1_ref[...] = xli[:, HID // 4:]
    xr_ref[...] = (
        jnp.dot(xb, wr_ref[...], preferred_element_type=jnp.float32)
        + b1_ref[...]
    )


def _mm1(x, W1l, W1rp, b1p):
    bm = 1000
    return pl.pallas_call(
        _mm1_body,
        grid=(N // bm,),
        in_specs=[
            pl.BlockSpec((bm, F), lambda i: (i, 0)),
            pl.BlockSpec((F, HID), lambda i: (0, 0)),
            pl.BlockSpec((F, HID), lambda i: (0, 0)),
            pl.BlockSpec((1, HID), lambda i: (0, 0)),
        ],
        out_specs=[
            pl.BlockSpec((bm, HID // 4), lambda i: (i, 0)),
            pl.BlockSpec((bm, HID // 4), lambda i: (i, 0)),
            pl.BlockSpec((bm, HID), lambda i: (i, 0)),
        ],
        out_shape=[
            jax.ShapeDtypeStruct((NROW, HID // 4), jnp.int32),
            jax.ShapeDtypeStruct((NROW, HID // 4), jnp.int32),
            jax.ShapeDtypeStruct((N, HID), jnp.float32),
        ],
    )(x, W1l, W1rp, b1p.reshape(1, HID))


def _sc_agg_body(wi, npass, with_deg, *refs):
    wf = 2 * wi
    tbls = refs[:npass]
    (srch, dsth, wh, zx) = refs[npass:npass + 4]
    aggs = refs[npass + 4:npass + 4 + npass]
    (src_v, dst_v, w_v, rowsi0, rowsi1, rowsf0, rowsf1,
     tbl_sh, accx, semg0, semg1, sems0, sems1) = refs[npass + 4 + npass:]
    c = lax.axis_index("c")
    s = lax.axis_index("s")
    wid = s * NC + c
    r0 = pl.multiple_of(s * RPT, 8)

    himask = jnp.full((16,), -65536, jnp.int32)

    if with_deg:
        def init_ones(i, _):
            rowsf0[i, pl.ds(wf, 16)] = jnp.full((16,), 1.0, jnp.float32)
            rowsf1[i, pl.ds(wf, 16)] = jnp.full((16,), 1.0, jnp.float32)
            return 0
        lax.fori_loop(0, CH, init_ones, 0)

    def mult(g, rowsi_v, rowsf_v):
        def grp(q, _):
            wv = w_v[g, pl.ds(q * 16, 16)]
            for l in range(16):
                bw = lax.gather(
                    wv, jnp.full((16, 1), l, jnp.int32),
                    lax.GatherDimensionNumbers(
                        offset_dims=(), collapsed_slice_dims=(0,),
                        start_index_map=(0,)),
                    (1,), mode=lax.GatherScatterMode.PROMISE_IN_BOUNDS)
                e = q * 16 + l
                for j in range(wi // 16):
                    v = rowsi_v[e, pl.ds(j * 16, 16)]
                    lo = plsc.bitcast(lax.shift_left(v, 16), jnp.float32)
                    hi = plsc.bitcast(lax.bitwise_and(v, himask),
                                      jnp.float32)
                    rowsf_v[e, pl.ds(32 * j, 16)] = lo * bw
                    rowsf_v[e, pl.ds(32 * j + 16, 16)] = hi * bw
            return 0
        lax.fori_loop(0, CH // 16, grp, 0)

    for p in range(npass):
        pltpu.sync_copy(zx.at[pl.ds(r0, RPT)], accx.at[pl.ds(r0, RPT)])
        pltpu.sync_copy(tbls[p].at[pl.ds(r0, RPT)],
                        tbl_sh.at[pl.ds(r0, RPT)])
        plsc.subcore_barrier()

        def block(b, _):
            crow = wid * CH_PER_W + b * IBLK
            pltpu.sync_copy(srch.at[pl.ds(crow, IBLK)], src_v)
            pltpu.sync_copy(dsth.at[pl.ds(crow, IBLK)], dst_v)
            pltpu.sync_copy(wh.at[pl.ds(crow, IBLK)], w_v)
            pltpu.async_copy(tbl_sh.at[src_v.at[0]], rowsi0, semg0)
            pltpu.async_copy(tbl_sh.at[src_v.at[1]], rowsi1, semg1)

            def pair(i, _):
                g0 = i * 2
                pltpu.make_async_copy(
                    tbl_sh.at[src_v.at[g0]], rowsi0, semg0).wait()

                @pl.when(i > 0)
                def _():
                    pltpu.make_async_copy(
                        rowsf0, accx.at[dst_v.at[g0 - 2]], sems0).wait()
                mult(g0, rowsi0, rowsf0)

                @pl.when(g0 + 2 < IBLK)
                def _():
                    pltpu.async_copy(
                        tbl_sh.at[src_v.at[g0 + 2]], rowsi0, semg0)
                pltpu.async_copy(
                    rowsf0, accx.at[dst_v.at[g0]], sems0, add=True)

                pltpu.make_async_copy(
                    tbl_sh.at[src_v.at[g0 + 1]], rowsi1, semg1).wait()

                @pl.when(i > 0)
                def _():
                    pltpu.make_async_copy(
                        rowsf1, accx.at[dst_v.at[g0 - 1]], sems1).wait()
                mult(g0 + 1, rowsi1, rowsf1)

                @pl.when(g0 + 3 < IBLK)
                def _():
                    pltpu.async_copy(
                        tbl_sh.at[src_v.at[g0 + 3]], rowsi1, semg1)
                pltpu.async_copy(
                    rowsf1, accx.at[dst_v.at[g0 + 1]], sems1, add=True)
                return 0
            lax.fori_loop(0, IBLK // 2, pair, 0)
            pltpu.make_async_copy(
                rowsf0, accx.at[dst_v.at[IBLK - 2]], sems0).wait()
            pltpu.make_async_copy(
                rowsf1, accx.at[dst_v.at[IBLK - 1]], sems1).wait()
            return 0
        lax.fori_loop(0, NBLK, block, 0)
        plsc.subcore_barrier()

        pltpu.sync_copy(accx.at[pl.ds(r0, RPT)], aggs[p].at[c, pl.ds(r0, RPT)])


def _sc_agg(wi, npass, with_deg):
    mesh = plsc.VectorSubcoreMesh(core_axis_name="c", subcore_axis_name="s")
    wa = 2 * wi + (16 if with_deg else 0)
    out_type = [jax.ShapeDtypeStruct((NC, NROW, wa), jnp.float32)
                for _ in range(npass)]
    scratch = [
        pltpu.VMEM((IBLK, CH), jnp.int32),
        pltpu.VMEM((IBLK, CH), jnp.int32),
        pltpu.VMEM((IBLK, CH), jnp.float32),
        pltpu.VMEM((CH, wi), jnp.int32),
        pltpu.VMEM((CH, wi), jnp.int32),
        pltpu.VMEM((CH, wa), jnp.float32),
        pltpu.VMEM((CH, wa), jnp.float32),
        pltpu.VMEM_SHARED((NROW, wi), jnp.int32),
        pltpu.VMEM_SHARED((NROW, wa), jnp.float32),
        pltpu.SemaphoreType.DMA,
        pltpu.SemaphoreType.DMA,
        pltpu.SemaphoreType.DMA,
        pltpu.SemaphoreType.DMA,
    ]
    return pl.kernel(
        functools.partial(_sc_agg_body, wi, npass, with_deg),
        out_type=out_type,
        mesh=mesh,
        scratch_types=scratch,
        compiler_params=pltpu.CompilerParams(
            use_tc_tiling_on_sc=False, needs_layout_passes=False),
    )


def _mid_body(a00_ref, a01_ref, a10_ref, a11_ref, xr_ref,
              wl_ref, wr_ref, b2_ref, hl_ref, hr_ref):
    aggl = a00_ref[...] + a01_ref[...]
    aggh = a10_ref[...] + a11_ref[...]
    agg = jnp.concatenate([aggl[:, :HID // 2], aggh[:, :HID // 2]], axis=1)
    deg = aggl[:, HID // 2:HID // 2 + 1]
    rdeg = 1.0 / jnp.maximum(deg, 1.0)
    h = jnp.maximum(agg * rdeg + xr_ref[...], 0.0)
    hl = jnp.dot(h, wl_ref[...], preferred_element_type=jnp.float32)
    hl_ref[...] = _tc_pack(hl)
    hr_ref[...] = (
        jnp.dot(h, wr_ref[...], preferred_element_type=jnp.float32)
        + b2_ref[...]
    )


def _mid(a00, a01, a10, a11, xr, W2lp, W2rp, b2p):
    bm = 1000
    wa = HID // 2 + 16
    return pl.pallas_call(
        _mid_body,
        grid=(N // bm,),
        in_specs=[
            pl.BlockSpec((bm, wa), lambda i: (i, 0)),
            pl.BlockSpec((bm, wa), lambda i: (i, 0)),
            pl.BlockSpec((bm, wa), lambda i: (i, 0)),
            pl.BlockSpec((bm, wa), lambda i: (i, 0)),
            pl.BlockSpec((bm, HID), lambda i: (i, 0)),
            pl.BlockSpec((HID, CP), lambda i: (0, 0)),
            pl.BlockSpec((HID, CP), lambda i: (0, 0)),
            pl.BlockSpec((1, CP), lambda i: (0, 0)),
        ],
        out_specs=[
            pl.BlockSpec((bm, CP // 2), lambda i: (i, 0)),
            pl.BlockSpec((bm, CP), lambda i: (i, 0)),
        ],
        out_shape=[
            jax.ShapeDtypeStruct((NROW, CP // 2), jnp.int32),
            jax.ShapeDtypeStruct((N, CP), jnp.float32),
        ],
    )(a00, a01, a10, a11, xr, W2lp, W2rp, b2p)


def _fin_body(a0_ref, a1_ref, d0_ref, d1_ref, hr_ref, msk_ref, p_ref,
              out_ref):
    agg = a0_ref[...] + a1_ref[...]
    deg = d0_ref[:, CP:CP + 1] + d1_ref[:, CP:CP + 1]
    rdeg = 1.0 / jnp.maximum(deg, 1.0)
    logits = agg * rdeg + hr_ref[...]
    masked = logits + msk_ref[...]
    m = jnp.max(masked, axis=1, keepdims=True)
    lse = jnp.log(jnp.sum(jnp.exp(masked - m), axis=1, keepdims=True)) + m
    out_ref[...] = jnp.dot(logits - lse, p_ref[...],
                           preferred_element_type=jnp.float32)


def _fin(a0, a1, d0, d1, hr, msk, pmat):
    bm = 1000
    return pl.pallas_call(
        _fin_body,
        grid=(N // bm,),
        in_specs=[
            pl.BlockSpec((bm, CP), lambda i: (i, 0)),
            pl.BlockSpec((bm, CP), lambda i: (i, 0)),
            pl.BlockSpec((bm, CP + 16), lambda i: (i, 0)),
            pl.BlockSpec((bm, CP + 16), lambda i: (i, 0)),
            pl.BlockSpec((bm, CP), lambda i: (i, 0)),
            pl.BlockSpec((1, CP), lambda i: (0, 0)),
            pl.BlockSpec((CP, 48), lambda i: (0, 0)),
        ],
        out_specs=pl.BlockSpec((bm, 48), lambda i: (i, 0)),
        out_shape=jax.ShapeDtypeStruct((N, 48), jnp.float32),
    )(a0, a1, d0, d1, hr, msk, pmat)


@jax.jit
def kernel(x, edge_index, edge_weight, W1l, W1r, b1, W2l, W2r, b2):
    src = edge_index[0]
    dst = edge_index[1]
    padw = EPW - ERW
    dpad = jnp.broadcast_to(
        DUMP0 + (jnp.arange(padw, dtype=jnp.int32) % 64), (NW, padw))
    srcp = jnp.concatenate(
        [src.reshape(NW, ERW), jnp.zeros((NW, padw), jnp.int32)],
        axis=1).reshape(EPAD // CH, CH)
    dstp = jnp.concatenate(
        [dst.reshape(NW, ERW), dpad], axis=1).reshape(EPAD // CH, CH)
    wp = jnp.concatenate(
        [edge_weight.reshape(NW, ERW), jnp.zeros((NW, padw), jnp.float32)],
        axis=1).reshape(EPAD // CH, CH)

    zx80 = jnp.zeros((NROW, CP + 16), jnp.float32)
    zx64 = jnp.zeros((NROW, CP), jnp.float32)

    W1rp = W1r[:, PERM1]
    b1p = b1[PERM1]
    W2l64 = jnp.pad(W2l, ((0, 0), (0, CP - C)))
    W2r64 = jnp.pad(W2r, ((0, 0), (0, CP - C)))
    W2lp = W2l64[PERM1, :]
    W2rp = W2r64[PERM1, :][:, PERM2]
    b2p = jnp.pad(b2, (0, CP - C))[PERM2].reshape(1, CP)
    msk = jnp.where(PERM2 < C, 0.0, -1e30).astype(jnp.float32).reshape(1, CP)
    pmat = (PERM2[:, None] == np.arange(48)[None, :]).astype(np.float32)

    tbl0, tbl1, xr = _mm1(x, W1l, W1rp, b1p)
    agg0, agg1 = _sc_agg(HID // 4, 2, True)(
        tbl0, tbl1, srcp, dstp, wp, zx80)
    hlt, hr = _mid(agg0[0], agg0[1], agg1[0], agg1[1],
                   xr, W2lp, W2rp, b2p)
    (agg2p,) = _sc_agg(CP // 2, 1, False)(hlt, srcp, dstp, wp, zx64)
    out = _fin(agg2p[0], agg2p[1], agg0[0], agg0[1], hr, msk,
               jnp.asarray(pmat))
    return out[:, :C]


# speedup vs baseline: 2.1925x; 1.4114x over previous
import functools

import numpy as np
import jax
import jax.numpy as jnp
from jax import lax
from jax.experimental import pallas as pl
from jax.experimental.pallas import tpu as pltpu
from jax.experimental.pallas import tpu_sc as plsc

N = 10000
F = 128
HID = 128
C = 40
CP = 64
E = 320000

NC = 2
NS = 16
NW = NC * NS
CH = 128
NROW = 10112
RPT = NROW // NS
DUMP0 = 10048
EPW = 10240
CH_PER_W = EPW // CH
IBLK = 16
NBLK = CH_PER_W // IBLK
EPAD = EPW * NW
ERW = E // NW


def _unpack_perm(width):
    half = width // 2
    perm = np.zeros((width,), np.int32)
    for j in range(half // 16):
        for l in range(16):
            perm[32 * j + l] = 16 * j + l
            perm[32 * j + 16 + l] = half + 16 * j + l
    return perm

PERM1 = _unpack_perm(HID)
PERM2 = _unpack_perm(CP)


def _tc_pack(a):
    k = a.shape[1] // 2
    bits = jax.lax.bitcast_convert_type(a, jnp.int32) + 0x8000
    lo = jax.lax.shift_right_logical(bits[:, :k], 16)
    hi = jax.lax.bitwise_and(bits[:, k:], -65536)
    return jax.lax.bitwise_or(lo, hi)


def _mm1_body(x_ref, wl_ref, wr_ref, b1_ref, t0_ref, t1_ref, xr_ref):
    xb = x_ref[...]
    xl = jnp.dot(xb, wl_ref[...], preferred_element_type=jnp.float32)
    xli = _tc_pack(xl)
    t0_ref[...] = xli[:, :HID // 4]
    t1_ref[...] = xli[:, HID // 4:]
    xr_ref[...] = (
        jnp.dot(xb, wr_ref[...], preferred_element_type=jnp.float32)
        + b1_ref[...]
    )


def _mm1(x, W1l, W1rp, b1p):
    bm = 1000
    return pl.pallas_call(
        _mm1_body,
        grid=(N // bm,),
        in_specs=[
            pl.BlockSpec((bm, F), lambda i: (i, 0)),
            pl.BlockSpec((F, HID), lambda i: (0, 0)),
            pl.BlockSpec((F, HID), lambda i: (0, 0)),
            pl.BlockSpec((1, HID), lambda i: (0, 0)),
        ],
        out_specs=[
            pl.BlockSpec((bm, HID // 4), lambda i: (i, 0)),
            pl.BlockSpec((bm, HID // 4), lambda i: (i, 0)),
            pl.BlockSpec((bm, HID), lambda i: (i, 0)),
        ],
        out_shape=[
            jax.ShapeDtypeStruct((NROW, HID // 4), jnp.int32),
            jax.ShapeDtypeStruct((NROW, HID // 4), jnp.int32),
            jax.ShapeDtypeStruct((N, HID), jnp.float32),
        ],
    )(x, W1l, W1rp, b1p.reshape(1, HID))


def _sc_agg_body(wi, npass, with_deg, *refs):
    wf = 2 * wi
    tbls = refs[:npass]
    (srch, dsth, wh, zx) = refs[npass:npass + 4]
    aggs = refs[npass + 4:npass + 4 + npass]
    (src_v, dst_v, w_v, rowsi0, rowsi1, rowsf0, rowsf1,
     tbl_sh, accx, semg0, semg1, sems0, sems1) = refs[npass + 4 + npass:]
    c = lax.axis_index("c")
    s = lax.axis_index("s")
    wid = s * NC + c
    r0 = pl.multiple_of(s * RPT, 8)

    himask = jnp.full((16,), -65536, jnp.int32)

    if with_deg:
        def init_ones(i, _):
            rowsf0[i, pl.ds(wf, 16)] = jnp.full((16,), 1.0, jnp.float32)
            rowsf1[i, pl.ds(wf, 16)] = jnp.full((16,), 1.0, jnp.float32)
            return 0
        lax.fori_loop(0, CH, init_ones, 0)

    def mult(g, rowsi_v, rowsf_v):
        for q in range(CH // 16):
            wv = w_v[g, pl.ds(q * 16, 16)]
            for l in range(16):
                bw = lax.gather(
                    wv, jnp.full((16, 1), l, jnp.int32),
                    lax.GatherDimensionNumbers(
                        offset_dims=(), collapsed_slice_dims=(0,),
                        start_index_map=(0,)),
                    (1,), mode=lax.GatherScatterMode.PROMISE_IN_BOUNDS)
                e = q * 16 + l
                for j in range(wi // 16):
                    v = rowsi_v[e, pl.ds(j * 16, 16)]
                    lo = plsc.bitcast(lax.shift_left(v, 16), jnp.float32)
                    hi = plsc.bitcast(lax.bitwise_and(v, himask),
                                      jnp.float32)
                    rowsf_v[e, pl.ds(32 * j, 16)] = lo * bw
                    rowsf_v[e, pl.ds(32 * j + 16, 16)] = hi * bw

    for p in range(npass):
        pltpu.sync_copy(zx.at[pl.ds(r0, RPT)], accx.at[pl.ds(r0, RPT)])
        pltpu.sync_copy(tbls[p].at[pl.ds(r0, RPT)],
                        tbl_sh.at[pl.ds(r0, RPT)])
        plsc.subcore_barrier()

        def block(b, _):
            crow = wid * CH_PER_W + b * IBLK
            pltpu.sync_copy(srch.at[pl.ds(crow, IBLK)], src_v)
            pltpu.sync_copy(dsth.at[pl.ds(crow, IBLK)], dst_v)
            pltpu.sync_copy(wh.at[pl.ds(crow, IBLK)], w_v)
            pltpu.async_copy(tbl_sh.at[src_v.at[0]], rowsi0, semg0)
            pltpu.async_copy(tbl_sh.at[src_v.at[1]], rowsi1, semg1)

            def pair(i, _):
                g0 = i * 2
                pltpu.make_async_copy(
                    tbl_sh.at[src_v.at[g0]], rowsi0, semg0).wait()

                @pl.when(i > 0)
                def _():
                    pltpu.make_async_copy(
                        rowsf0, accx.at[dst_v.at[g0 - 2]], sems0).wait()
                mult(g0, rowsi0, rowsf0)

                @pl.when(g0 + 2 < IBLK)
                def _():
                    pltpu.async_copy(
                        tbl_sh.at[src_v.at[g0 + 2]], rowsi0, semg0)
                pltpu.async_copy(
                    rowsf0, accx.at[dst_v.at[g0]], sems0, add=True)

                pltpu.make_async_copy(
                    tbl_sh.at[src_v.at[g0 + 1]], rowsi1, semg1).wait()

                @pl.when(i > 0)
                def _():
                    pltpu.make_async_copy(
                        rowsf1, accx.at[dst_v.at[g0 - 1]], sems1).wait()
                mult(g0 + 1, rowsi1, rowsf1)

                @pl.when(g0 + 3 < IBLK)
                def _():
                    pltpu.async_copy(
                        tbl_sh.at[src_v.at[g0 + 3]], rowsi1, semg1)
                pltpu.async_copy(
                    rowsf1, accx.at[dst_v.at[g0 + 1]], sems1, add=True)
                return 0
            lax.fori_loop(0, IBLK // 2, pair, 0)
            pltpu.make_async_copy(
                rowsf0, accx.at[dst_v.at[IBLK - 2]], sems0).wait()
            pltpu.make_async_copy(
                rowsf1, accx.at[dst_v.at[IBLK - 1]], sems1).wait()
            return 0
        lax.fori_loop(0, NBLK, block, 0)
        plsc.subcore_barrier()

        pltpu.sync_copy(accx.at[pl.ds(r0, RPT)], aggs[p].at[c, pl.ds(r0, RPT)])


def _sc_agg(wi, npass, with_deg):
    mesh = plsc.VectorSubcoreMesh(core_axis_name="c", subcore_axis_name="s")
    wa = 2 * wi + (16 if with_deg else 0)
    out_type = [jax.ShapeDtypeStruct((NC, NROW, wa), jnp.float32)
                for _ in range(npass)]
    scratch = [
        pltpu.VMEM((IBLK, CH), jnp.int32),
        pltpu.VMEM((IBLK, CH), jnp.int32),
        pltpu.VMEM((IBLK, CH), jnp.float32),
        pltpu.VMEM((CH, wi), jnp.int32),
        pltpu.VMEM((CH, wi), jnp.int32),
        pltpu.VMEM((CH, wa), jnp.float32),
        pltpu.VMEM((CH, wa), jnp.float32),
        pltpu.VMEM_SHARED((NROW, wi), jnp.int32),
        pltpu.VMEM_SHARED((NROW, wa), jnp.float32),
        pltpu.SemaphoreType.DMA,
        pltpu.SemaphoreType.DMA,
        pltpu.SemaphoreType.DMA,
        pltpu.SemaphoreType.DMA,
    ]
    return pl.kernel(
        functools.partial(_sc_agg_body, wi, npass, with_deg),
        out_type=out_type,
        mesh=mesh,
        scratch_types=scratch,
        compiler_params=pltpu.CompilerParams(
            use_tc_tiling_on_sc=False, needs_layout_passes=False),
    )


def _mid_body(a00_ref, a01_ref, a10_ref, a11_ref, xr_ref,
              wl_ref, wr_ref, b2_ref, hl_ref, hr_ref):
    aggl = a00_ref[...] + a01_ref[...]
    aggh = a10_ref[...] + a11_ref[...]
    agg = jnp.concatenate([aggl[:, :HID // 2], aggh[:, :HID // 2]], axis=1)
    deg = aggl[:, HID // 2:HID // 2 + 1]
    rdeg = 1.0 / jnp.maximum(deg, 1.0)
    h = jnp.maximum(agg * rdeg + xr_ref[...], 0.0)
    hl = jnp.dot(h, wl_ref[...], preferred_element_type=jnp.float32)
    hl_ref[...] = _tc_pack(hl)
    hr_ref[...] = (
        jnp.dot(h, wr_ref[...], preferred_element_type=jnp.float32)
        + b2_ref[...]
    )


def _mid(a00, a01, a10, a11, xr, W2lp, W2rp, b2p):
    bm = 1000
    wa = HID // 2 + 16
    return pl.pallas_call(
        _mid_body,
        grid=(N // bm,),
        in_specs=[
            pl.BlockSpec((bm, wa), lambda i: (i, 0)),
            pl.BlockSpec((bm, wa), lambda i: (i, 0)),
            pl.BlockSpec((bm, wa), lambda i: (i, 0)),
            pl.BlockSpec((bm, wa), lambda i: (i, 0)),
            pl.BlockSpec((bm, HID), lambda i: (i, 0)),
            pl.BlockSpec((HID, CP), lambda i: (0, 0)),
            pl.BlockSpec((HID, CP), lambda i: (0, 0)),
            pl.BlockSpec((1, CP), lambda i: (0, 0)),
        ],
        out_specs=[
            pl.BlockSpec((bm, CP // 2), lambda i: (i, 0)),
            pl.BlockSpec((bm, CP), lambda i: (i, 0)),
        ],
        out_shape=[
            jax.ShapeDtypeStruct((NROW, CP // 2), jnp.int32),
            jax.ShapeDtypeStruct((N, CP), jnp.float32),
        ],
    )(a00, a01, a10, a11, xr, W2lp, W2rp, b2p)


def _fin_body(a0_ref, a1_ref, d0_ref, d1_ref, hr_ref, msk_ref, p_ref,
              out_ref):
    agg = a0_ref[...] + a1_ref[...]
    deg = d0_ref[:, CP:CP + 1] + d1_ref[:, CP:CP + 1]
    rdeg = 1.0 / jnp.maximum(deg, 1.0)
    logits = agg * rdeg + hr_ref[...]
    masked = logits + msk_ref[...]
    m = jnp.max(masked, axis=1, keepdims=True)
    lse = jnp.log(jnp.sum(jnp.exp(masked - m), axis=1, keepdims=True)) + m
    out_ref[...] = jnp.dot(logits - lse, p_ref[...],
                           preferred_element_type=jnp.float32)


def _fin(a0, a1, d0, d1, hr, msk, pmat):
    bm = 1000
    return pl.pallas_call(
        _fin_body,
        grid=(N // bm,),
        in_specs=[
            pl.BlockSpec((bm, CP), lambda i: (i, 0)),
            pl.BlockSpec((bm, CP), lambda i: (i, 0)),
            pl.BlockSpec((bm, CP + 16), lambda i: (i, 0)),
            pl.BlockSpec((bm, CP + 16), lambda i: (i, 0)),
            pl.BlockSpec((bm, CP), lambda i: (i, 0)),
            pl.BlockSpec((1, CP), lambda i: (0, 0)),
            pl.BlockSpec((CP, 48), lambda i: (0, 0)),
        ],
        out_specs=pl.BlockSpec((bm, 48), lambda i: (i, 0)),
        out_shape=jax.ShapeDtypeStruct((N, 48), jnp.float32),
    )(a0, a1, d0, d1, hr, msk, pmat)


@jax.jit
def kernel(x, edge_index, edge_weight, W1l, W1r, b1, W2l, W2r, b2):
    src = edge_index[0]
    dst = edge_index[1]
    padw = EPW - ERW
    dpad = jnp.broadcast_to(
        DUMP0 + (jnp.arange(padw, dtype=jnp.int32) % 64), (NW, padw))
    srcp = jnp.concatenate(
        [src.reshape(NW, ERW), jnp.zeros((NW, padw), jnp.int32)],
        axis=1).reshape(EPAD // CH, CH)
    dstp = jnp.concatenate(
        [dst.reshape(NW, ERW), dpad], axis=1).reshape(EPAD // CH, CH)
    wp = jnp.concatenate(
        [edge_weight.reshape(NW, ERW), jnp.zeros((NW, padw), jnp.float32)],
        axis=1).reshape(EPAD // CH, CH)

    zx80 = jnp.zeros((NROW, CP + 16), jnp.float32)
    zx64 = jnp.zeros((NROW, CP), jnp.float32)

    W1rp = W1r[:, PERM1]
    b1p = b1[PERM1]
    W2l64 = jnp.pad(W2l, ((0, 0), (0, CP - C)))
    W2r64 = jnp.pad(W2r, ((0, 0), (0, CP - C)))
    W2lp = W2l64[PERM1, :]
    W2rp = W2r64[PERM1, :][:, PERM2]
    b2p = jnp.pad(b2, (0, CP - C))[PERM2].reshape(1, CP)
    msk = jnp.where(PERM2 < C, 0.0, -1e30).astype(jnp.float32).reshape(1, CP)
    pmat = (PERM2[:, None] == np.arange(48)[None, :]).astype(np.float32)

    tbl0, tbl1, xr = _mm1(x, W1l, W1rp, b1p)
    agg0, agg1 = _sc_agg(HID // 4, 2, True)(
        tbl0, tbl1, srcp, dstp, wp, zx80)
    hlt, hr = _mid(agg0[0], agg0[1], agg1[0], agg1[1],
                   xr, W2lp, W2rp, b2p)
    (agg2p,) = _sc_agg(CP // 2, 1, False)(hlt, srcp, dstp, wp, zx64)
    out = _fin(agg2p[0], agg2p[1], agg0[0], agg0[1], hr, msk,
               jnp.asarray(pmat))
    return out[:, :C]
